# baseline (speedup 1.0000x reference)
import functools

import jax
import jax.numpy as jnp
from jax import lax
from jax.experimental import pallas as pl
from jax.experimental.pallas import tpu as pltpu
from jax.experimental.pallas import tpu_sc as plsc

_B = 4096
_F = 26
_V = 100000
_D = 32
_NC, _NS = 2, 16
_NW = _NC * _NS
_RPW = _B // _NW
_LPW = _RPW * _F
_D2 = _D // 2
_PPW = _LPW * _D2
_CH = 128
_SEG = 2 * _CH
_NSEG = 8


@functools.cache
def _build_sc_gather():
    mesh = plsc.VectorSubcoreMesh(core_axis_name="c", subcore_axis_name="s",
                                  num_cores=_NC, num_subcores=_NS)
    return functools.partial(
        pl.kernel,
        out_type=jax.ShapeDtypeStruct((_B * _F * _D2,), jnp.int32),
        mesh=mesh,
        scratch_types=[
            pltpu.VMEM((_LPW,), jnp.int32),
            pltpu.VMEM((_NSEG * _SEG,), jnp.int32),
            pltpu.VMEM((_PPW,), jnp.int32),
            pltpu.SemaphoreType.DMA,
        ],
        compiler_params=pltpu.CompilerParams(use_tc_tiling_on_sc=False),
    )(_sc_gather_body)


def _sc_gather_body(emb_hbm, cat_hbm, out_hbm, cat_v, ring_v, val_v, sem):
    wid = lax.axis_index("s") * _NC + lax.axis_index("c")
    base = wid * _LPW
    pltpu.sync_copy(cat_hbm.at[pl.ds(base, _LPW)], cat_v)

    lane = lax.iota(jnp.int32, 16)
    laneV = lane * _V

    def chunk_copy(c, slot):
        return pltpu.make_async_copy(
            emb_hbm.at[ring_v.at[pl.ds(slot * _CH, _CH)]],
            val_v.at[pl.ds(c * _CH, _CH)],
            sem,
        )

    n_outer = _LPW // 16

    def outer(i, _):
        seg = lax.rem(i, _NSEG)

        @pl.when(i >= _NSEG)
        def _wait_prev():
            prev = i - _NSEG
            pseg = lax.rem(prev, _NSEG)
            for k in range(2):
                chunk_copy(prev * 2 + k, pseg * 2 + k).wait()

        cat16 = cat_v[pl.ds(i * 16, 16)]
        f16 = (i * 16 + lane) % _F
        base16 = f16 * (_D2 * _V) + cat16
        gdn = lax.GatherDimensionNumbers(
            offset_dims=(), collapsed_slice_dims=(0,), start_index_map=(0,))
        for j in range(16):
            bj = lax.gather(base16, jnp.full((16, 1), j, jnp.int32), gdn,
                            slice_sizes=(1,),
                            mode=lax.GatherScatterMode.PROMISE_IN_BOUNDS)
            ring_v[pl.ds(seg * _SEG + j * 16, 16)] = bj + laneV
        for k in range(2):
            pltpu.async_copy(
                emb_hbm.at[ring_v.at[pl.ds(seg * _SEG + k * _CH, _CH)]],
                val_v.at[pl.ds((i * 2 + k) * _CH, _CH)],
                sem,
            )
        return 0

    lax.fori_loop(0, n_outer, outer, 0)

    def drain(j, _):
        c = (n_outer - _NSEG) * 2 + j
        slot = lax.rem(c, _NSEG * 2)
        chunk_copy(c, slot).wait()
        return 0

    lax.fori_loop(0, _NSEG * 2, drain, 0)
    pltpu.sync_copy(val_v, out_hbm.at[pl.ds(wid * _PPW, _PPW)])


_BT = 512


def _mlp_body(xe_ref, xc_ref, w0e_ref, w0c_ref, b0_ref, w1_ref, b1_ref,
              w2_ref, b2_ref, w3_ref, b3_ref, o_ref):
    dn = (((1,), (1,)), ((), ()))
    h = lax.dot_general(xe_ref[...], w0e_ref[...], dn,
                        preferred_element_type=jnp.float32)
    h += lax.dot_general(xc_ref[...], w0c_ref[...], dn,
                         preferred_element_type=jnp.float32)
    h = jnp.maximum(h + b0_ref[...], 0.0)
    h = lax.dot_general(h, w1_ref[...], dn, preferred_element_type=jnp.float32)
    h = jnp.maximum(h + b1_ref[...], 0.0)
    h = lax.dot_general(h, w2_ref[...], dn, preferred_element_type=jnp.float32)
    h = jnp.maximum(h + b2_ref[...], 0.0)
    o = lax.dot_general(h, w3_ref[...], dn, preferred_element_type=jnp.float32)
    o_ref[...] = jax.nn.sigmoid(o[:, :1] + b3_ref[0, 0])


def _mlp(x_emb, cont, w0e, w0c, b0, w1, b1, w2, b2, w3, b3):
    grid = (_B // _BT,)
    full = lambda shape: pl.BlockSpec(shape, lambda i: (0, 0))
    return pl.pallas_call(
        _mlp_body,
        grid=grid,
        in_specs=[
            pl.BlockSpec((_BT, _F * _D), lambda i: (i, 0)),
            pl.BlockSpec((_BT, 13), lambda i: (i, 0)),
            full(w0e.shape), full(w0c.shape), full(b0.shape),
            full(w1.shape), full(b1.shape),
            full(w2.shape), full(b2.shape),
            full(w3.shape), full(b3.shape),
        ],
        out_specs=pl.BlockSpec((_BT, 1), lambda i: (i, 0)),
        out_shape=jax.ShapeDtypeStruct((_B, 1), jnp.float32),
    )(x_emb, cont, w0e, w0c, b0, w1, b1, w2, b2, w3, b3)


def kernel(cont_data, cat_data, emb_tables, W0, b0, W1, b1, W2, b2, W3, b3):
    emb_bf = (emb_tables.astype(jnp.bfloat16)
              .reshape(_F, _V, _D2, 2).transpose(0, 2, 1, 3))
    emb_lin = lax.bitcast_convert_type(emb_bf, jnp.int32).reshape(_F * _D2 * _V)
    cat_flat = cat_data.reshape(_B * _F)
    x_i32 = _build_sc_gather()(emb_lin, cat_flat)
    x_emb = lax.bitcast_convert_type(x_i32, jnp.bfloat16).reshape(_B, _F * _D)
    w0e = W0[:, : _F * _D].astype(jnp.bfloat16)
    w0c = W0[:, _F * _D:]
    w3p = jnp.pad(W3, ((0, 127), (0, 0)))
    return _mlp(x_emb, cont_data,
                w0e, w0c, b0.reshape(1, -1),
                W1, b1.reshape(1, -1),
                W2, b2.reshape(1, -1),
                w3p, b3.reshape(1, -1))


# speedup vs baseline: 2.5948x; 2.5948x over previous
import functools

import jax
import jax.numpy as jnp
from jax import lax
from jax.experimental import pallas as pl
from jax.experimental.pallas import tpu as pltpu
from jax.experimental.pallas import tpu_sc as plsc

_B = 4096
_F = 26
_V = 100000
_D = 32
_NC, _NS = 2, 16
_NW = _NC * _NS
_RPW = _B // _NW
_LPW = _RPW * _F
_EPW = _LPW * _D
_CH = 128
_SEG = 4 * _CH
_NSEG = 8


@functools.cache
def _build_sc_gather():
    mesh = plsc.VectorSubcoreMesh(core_axis_name="c", subcore_axis_name="s",
                                  num_cores=_NC, num_subcores=_NS)
    return functools.partial(
        pl.kernel,
        out_type=jax.ShapeDtypeStruct((_B * _F * _D,), jnp.float32),
        mesh=mesh,
        scratch_types=[
            pltpu.VMEM((_LPW,), jnp.int32),
            pltpu.VMEM((_NSEG * _SEG,), jnp.int32),
            pltpu.VMEM((_EPW,), jnp.float32),
            pltpu.SemaphoreType.DMA,
        ],
        compiler_params=pltpu.CompilerParams(use_tc_tiling_on_sc=False),
    )(_sc_gather_body)


def _sc_gather_body(emb_hbm, cat_hbm, out_hbm, cat_v, ring_v, val_v, sem):
    wid = lax.axis_index("s") * _NC + lax.axis_index("c")
    base = wid * _LPW
    pltpu.sync_copy(cat_hbm.at[pl.ds(base, _LPW)], cat_v)

    lane = lax.iota(jnp.int32, 16)
    laneV = lane * _V

    def chunk_copy(c, slot):
        return pltpu.make_async_copy(
            emb_hbm.at[ring_v.at[pl.ds(slot * _CH, _CH)]],
            val_v.at[pl.ds(c * _CH, _CH)],
            sem,
        )

    n_outer = _LPW // 16

    def outer(i, _):
        seg = lax.rem(i, _NSEG)

        @pl.when(i >= _NSEG)
        def _wait_prev():
            prev = i - _NSEG
            pseg = lax.rem(prev, _NSEG)
            for k in range(4):
                chunk_copy(prev * 4 + k, pseg * 4 + k).wait()

        cat16 = cat_v[pl.ds(i * 16, 16)]
        f16 = (i * 16 + lane) % _F
        base16 = f16 * (_D * _V) + cat16
        gdn = lax.GatherDimensionNumbers(
            offset_dims=(), collapsed_slice_dims=(0,), start_index_map=(0,))
        for j in range(16):
            bj = lax.gather(base16, jnp.full((16, 1), j, jnp.int32), gdn,
                            slice_sizes=(1,),
                            mode=lax.GatherScatterMode.PROMISE_IN_BOUNDS)
            for h in range(2):
                ring_v[pl.ds(seg * _SEG + j * _D + h * 16, 16)] = (
                    bj + laneV + h * 16 * _V)
        for k in range(4):
            pltpu.async_copy(
                emb_hbm.at[ring_v.at[pl.ds(seg * _SEG + k * _CH, _CH)]],
                val_v.at[pl.ds((i * 4 + k) * _CH, _CH)],
                sem,
            )
        return 0

    lax.fori_loop(0, n_outer, outer, 0)

    def drain(j, _):
        c = (n_outer - _NSEG) * 4 + j
        slot = lax.rem(c, _NSEG * 4)
        chunk_copy(c, slot).wait()
        return 0

    lax.fori_loop(0, _NSEG * 4, drain, 0)
    pltpu.sync_copy(val_v, out_hbm.at[pl.ds(wid * _EPW, _EPW)])


_BT = 512


def _mlp_body(xe_ref, xc_ref, w0e_ref, w0c_ref, b0_ref, w1_ref, b1_ref,
              w2_ref, b2_ref, w3_ref, b3_ref, o_ref):
    dn = (((1,), (1,)), ((), ()))
    h = lax.dot_general(xe_ref[...], w0e_ref[...], dn,
                        preferred_element_type=jnp.float32)
    h += lax.dot_general(xc_ref[...], w0c_ref[...], dn,
                         preferred_element_type=jnp.float32)
    h = jnp.maximum(h + b0_ref[...], 0.0)
    h = lax.dot_general(h, w1_ref[...], dn, preferred_element_type=jnp.float32)
    h = jnp.maximum(h + b1_ref[...], 0.0)
    h = lax.dot_general(h, w2_ref[...], dn, preferred_element_type=jnp.float32)
    h = jnp.maximum(h + b2_ref[...], 0.0)
    o = lax.dot_general(h, w3_ref[...], dn, preferred_element_type=jnp.float32)
    o_ref[...] = jax.nn.sigmoid(o[:, :1] + b3_ref[0, 0])


def _mlp(x_emb, cont, w0e, w0c, b0, w1, b1, w2, b2, w3, b3):
    grid = (_B // _BT,)
    full = lambda shape: pl.BlockSpec(shape, lambda i: (0, 0))
    return pl.pallas_call(
        _mlp_body,
        grid=grid,
        in_specs=[
            pl.BlockSpec((_BT, _F * _D), lambda i: (i, 0)),
            pl.BlockSpec((_BT, 13), lambda i: (i, 0)),
            full(w0e.shape), full(w0c.shape), full(b0.shape),
            full(w1.shape), full(b1.shape),
            full(w2.shape), full(b2.shape),
            full(w3.shape), full(b3.shape),
        ],
        out_specs=pl.BlockSpec((_BT, 1), lambda i: (i, 0)),
        out_shape=jax.ShapeDtypeStruct((_B, 1), jnp.float32),
    )(x_emb, cont, w0e, w0c, b0, w1, b1, w2, b2, w3, b3)


def kernel(cont_data, cat_data, emb_tables, W0, b0, W1, b1, W2, b2, W3, b3):
    emb_lin = emb_tables.transpose(0, 2, 1).reshape(_F * _D * _V)
    cat_flat = cat_data.reshape(_B * _F)
    x_emb = _build_sc_gather()(emb_lin, cat_flat).reshape(_B, _F * _D)
    w0e = W0[:, : _F * _D]
    w0c = W0[:, _F * _D:]
    w3p = jnp.pad(W3, ((0, 127), (0, 0)))
    return _mlp(x_emb, cont_data,
                w0e, w0c, b0.reshape(1, -1),
                W1, b1.reshape(1, -1),
                W2, b2.reshape(1, -1),
                w3p, b3.reshape(1, -1))


# speedup vs baseline: 2.6368x; 1.0162x over previous
import functools

import jax
import jax.numpy as jnp
from jax import lax
from jax.experimental import pallas as pl
from jax.experimental.pallas import tpu as pltpu
from jax.experimental.pallas import tpu_sc as plsc

_B = 4096
_F = 26
_V = 100000
_D = 32
_NC, _NS = 2, 16
_NW = _NC * _NS
_RPW = _B // _NW
_LPW = _RPW * _F
_EPW = _LPW * _D
_CH = 128
_SEG = 4 * _CH
_NSEG = 16


@functools.cache
def _build_sc_gather():
    mesh = plsc.VectorSubcoreMesh(core_axis_name="c", subcore_axis_name="s",
                                  num_cores=_NC, num_subcores=_NS)
    return functools.partial(
        pl.kernel,
        out_type=jax.ShapeDtypeStruct((_B * _F * _D,), jnp.float32),
        mesh=mesh,
        scratch_types=[
            pltpu.VMEM((_LPW,), jnp.int32),
            pltpu.VMEM((_NSEG * _SEG,), jnp.int32),
            pltpu.VMEM((_EPW,), jnp.float32),
            pltpu.SemaphoreType.DMA,
        ],
        compiler_params=pltpu.CompilerParams(use_tc_tiling_on_sc=False),
    )(_sc_gather_body)


def _sc_gather_body(emb_hbm, cat_hbm, out_hbm, cat_v, ring_v, val_v, sem):
    wid = lax.axis_index("s") * _NC + lax.axis_index("c")
    base = wid * _LPW
    pltpu.sync_copy(cat_hbm.at[pl.ds(base, _LPW)], cat_v)

    lane = lax.iota(jnp.int32, 16)
    laneV = lane * _V

    def chunk_copy(c, slot):
        return pltpu.make_async_copy(
            emb_hbm.at[ring_v.at[pl.ds(slot * _CH, _CH)]],
            val_v.at[pl.ds(c * _CH, _CH)],
            sem,
        )

    n_outer = _LPW // 16

    def outer(i, _):
        seg = lax.rem(i, _NSEG)

        @pl.when(i >= _NSEG)
        def _wait_prev():
            prev = i - _NSEG
            pseg = lax.rem(prev, _NSEG)
            for k in range(4):
                chunk_copy(prev * 4 + k, pseg * 4 + k).wait()

        cat16 = cat_v[pl.ds(i * 16, 16)]
        f16 = (i * 16 + lane) % _F
        base16 = f16 * (_D * _V) + cat16
        gdn = lax.GatherDimensionNumbers(
            offset_dims=(), collapsed_slice_dims=(0,), start_index_map=(0,))
        for j in range(16):
            bj = lax.gather(base16, jnp.full((16, 1), j, jnp.int32), gdn,
                            slice_sizes=(1,),
                            mode=lax.GatherScatterMode.PROMISE_IN_BOUNDS)
            for h in range(2):
                ring_v[pl.ds(seg * _SEG + j * _D + h * 16, 16)] = (
                    bj + laneV + h * 16 * _V)
        for k in range(4):
            pltpu.async_copy(
                emb_hbm.at[ring_v.at[pl.ds(seg * _SEG + k * _CH, _CH)]],
                val_v.at[pl.ds((i * 4 + k) * _CH, _CH)],
                sem,
            )
        return 0

    lax.fori_loop(0, n_outer, outer, 0)

    def drain(j, _):
        c = (n_outer - _NSEG) * 4 + j
        slot = lax.rem(c, _NSEG * 4)
        chunk_copy(c, slot).wait()
        return 0

    lax.fori_loop(0, _NSEG * 4, drain, 0)
    pltpu.sync_copy(val_v, out_hbm.at[pl.ds(wid * _EPW, _EPW)])


_BT = 512


def _mlp_body(xe_ref, xc_ref, w0e_ref, w0c_ref, b0_ref, w1_ref, b1_ref,
              w2_ref, b2_ref, w3_ref, b3_ref, o_ref):
    dn = (((1,), (1,)), ((), ()))
    h = lax.dot_general(xe_ref[...], w0e_ref[...], dn,
                        preferred_element_type=jnp.float32)
    h += lax.dot_general(xc_ref[...], w0c_ref[...], dn,
                         preferred_element_type=jnp.float32)
    h = jnp.maximum(h + b0_ref[...], 0.0)
    h = lax.dot_general(h, w1_ref[...], dn, preferred_element_type=jnp.float32)
    h = jnp.maximum(h + b1_ref[...], 0.0)
    h = lax.dot_general(h, w2_ref[...], dn, preferred_element_type=jnp.float32)
    h = jnp.maximum(h + b2_ref[...], 0.0)
    o = lax.dot_general(h, w3_ref[...], dn, preferred_element_type=jnp.float32)
    o_ref[...] = jax.nn.sigmoid(o[:, :1] + b3_ref[0, 0])


def _mlp(x_emb, cont, w0e, w0c, b0, w1, b1, w2, b2, w3, b3):
    grid = (_B // _BT,)
    full = lambda shape: pl.BlockSpec(shape, lambda i: (0, 0))
    return pl.pallas_call(
        _mlp_body,
        grid=grid,
        in_specs=[
            pl.BlockSpec((_BT, _F * _D), lambda i: (i, 0)),
            pl.BlockSpec((_BT, 13), lambda i: (i, 0)),
            full(w0e.shape), full(w0c.shape), full(b0.shape),
            full(w1.shape), full(b1.shape),
            full(w2.shape), full(b2.shape),
            full(w3.shape), full(b3.shape),
        ],
        out_specs=pl.BlockSpec((_BT, 1), lambda i: (i, 0)),
        out_shape=jax.ShapeDtypeStruct((_B, 1), jnp.float32),
    )(x_emb, cont, w0e, w0c, b0, w1, b1, w2, b2, w3, b3)


def kernel(cont_data, cat_data, emb_tables, W0, b0, W1, b1, W2, b2, W3, b3):
    emb_lin = emb_tables.transpose(0, 2, 1).reshape(_F * _D * _V)
    cat_flat = cat_data.reshape(_B * _F)
    x_emb = _build_sc_gather()(emb_lin, cat_flat).reshape(_B, _F * _D)
    w0e = W0[:, : _F * _D]
    w0c = W0[:, _F * _D:]
    w3p = jnp.pad(W3, ((0, 127), (0, 0)))
    return _mlp(x_emb, cont_data,
                w0e, w0c, b0.reshape(1, -1),
                W1, b1.reshape(1, -1),
                W2, b2.reshape(1, -1),
                w3p, b3.reshape(1, -1))


# speedup vs baseline: 2.6826x; 1.0174x over previous
import functools

import jax
import jax.numpy as jnp
from jax import lax
from jax.experimental import pallas as pl
from jax.experimental.pallas import tpu as pltpu
from jax.experimental.pallas import tpu_sc as plsc

_B = 4096
_F = 26
_V = 100000
_D = 32
_NC, _NS = 2, 16
_NW = _NC * _NS
_RPW = _B // _NW
_LPW = _RPW * _F
_EPW = _LPW * _D
_CH = 128
_SEG = 4 * _CH
_NSEG = 32


@functools.cache
def _build_sc_gather():
    mesh = plsc.VectorSubcoreMesh(core_axis_name="c", subcore_axis_name="s",
                                  num_cores=_NC, num_subcores=_NS)
    return functools.partial(
        pl.kernel,
        out_type=jax.ShapeDtypeStruct((_B * _F * _D,), jnp.float32),
        mesh=mesh,
        scratch_types=[
            pltpu.VMEM((_LPW,), jnp.int32),
            pltpu.VMEM((_NSEG * _SEG,), jnp.int32),
            pltpu.VMEM((_EPW,), jnp.float32),
            pltpu.SemaphoreType.DMA,
        ],
        compiler_params=pltpu.CompilerParams(use_tc_tiling_on_sc=False),
    )(_sc_gather_body)


def _sc_gather_body(emb_hbm, cat_hbm, out_hbm, cat_v, ring_v, val_v, sem):
    wid = lax.axis_index("s") * _NC + lax.axis_index("c")
    base = wid * _LPW
    pltpu.sync_copy(cat_hbm.at[pl.ds(base, _LPW)], cat_v)

    lane = lax.iota(jnp.int32, 16)
    laneV = lane * _V

    def chunk_copy(c, slot):
        return pltpu.make_async_copy(
            emb_hbm.at[ring_v.at[pl.ds(slot * _CH, _CH)]],
            val_v.at[pl.ds(c * _CH, _CH)],
            sem,
        )

    n_outer = _LPW // 16

    def outer(i, _):
        seg = lax.rem(i, _NSEG)

        @pl.when(i >= _NSEG)
        def _wait_prev():
            prev = i - _NSEG
            pseg = lax.rem(prev, _NSEG)
            for k in range(4):
                chunk_copy(prev * 4 + k, pseg * 4 + k).wait()

        cat16 = cat_v[pl.ds(i * 16, 16)]
        f16 = (i * 16 + lane) % _F
        base16 = f16 * (_D * _V) + cat16
        gdn = lax.GatherDimensionNumbers(
            offset_dims=(), collapsed_slice_dims=(0,), start_index_map=(0,))
        for j in range(16):
            bj = lax.gather(base16, jnp.full((16, 1), j, jnp.int32), gdn,
                            slice_sizes=(1,),
                            mode=lax.GatherScatterMode.PROMISE_IN_BOUNDS)
            for h in range(2):
                ring_v[pl.ds(seg * _SEG + j * _D + h * 16, 16)] = (
                    bj + laneV + h * 16 * _V)
        for k in range(4):
            pltpu.async_copy(
                emb_hbm.at[ring_v.at[pl.ds(seg * _SEG + k * _CH, _CH)]],
                val_v.at[pl.ds((i * 4 + k) * _CH, _CH)],
                sem,
            )
        return 0

    lax.fori_loop(0, n_outer, outer, 0)

    def drain(j, _):
        c = (n_outer - _NSEG) * 4 + j
        slot = lax.rem(c, _NSEG * 4)
        chunk_copy(c, slot).wait()
        return 0

    lax.fori_loop(0, _NSEG * 4, drain, 0)
    pltpu.sync_copy(val_v, out_hbm.at[pl.ds(wid * _EPW, _EPW)])


_BT = 512


def _mlp_body(xe_ref, xc_ref, w0e_ref, w0c_ref, b0_ref, w1_ref, b1_ref,
              w2_ref, b2_ref, w3_ref, b3_ref, o_ref):
    dn = (((1,), (1,)), ((), ()))
    h = lax.dot_general(xe_ref[...], w0e_ref[...], dn,
                        preferred_element_type=jnp.float32)
    h += lax.dot_general(xc_ref[...], w0c_ref[...], dn,
                         preferred_element_type=jnp.float32)
    h = jnp.maximum(h + b0_ref[...], 0.0)
    h = lax.dot_general(h, w1_ref[...], dn, preferred_element_type=jnp.float32)
    h = jnp.maximum(h + b1_ref[...], 0.0)
    h = lax.dot_general(h, w2_ref[...], dn, preferred_element_type=jnp.float32)
    h = jnp.maximum(h + b2_ref[...], 0.0)
    o = lax.dot_general(h, w3_ref[...], dn, preferred_element_type=jnp.float32)
    o_ref[...] = jax.nn.sigmoid(o[:, :1] + b3_ref[0, 0])


def _mlp(x_emb, cont, w0e, w0c, b0, w1, b1, w2, b2, w3, b3):
    grid = (_B // _BT,)
    full = lambda shape: pl.BlockSpec(shape, lambda i: (0, 0))
    return pl.pallas_call(
        _mlp_body,
        grid=grid,
        in_specs=[
            pl.BlockSpec((_BT, _F * _D), lambda i: (i, 0)),
            pl.BlockSpec((_BT, 13), lambda i: (i, 0)),
            full(w0e.shape), full(w0c.shape), full(b0.shape),
            full(w1.shape), full(b1.shape),
            full(w2.shape), full(b2.shape),
            full(w3.shape), full(b3.shape),
        ],
        out_specs=pl.BlockSpec((_BT, 1), lambda i: (i, 0)),
        out_shape=jax.ShapeDtypeStruct((_B, 1), jnp.float32),
    )(x_emb, cont, w0e, w0c, b0, w1, b1, w2, b2, w3, b3)


def kernel(cont_data, cat_data, emb_tables, W0, b0, W1, b1, W2, b2, W3, b3):
    emb_lin = emb_tables.transpose(0, 2, 1).reshape(_F * _D * _V)
    cat_flat = cat_data.reshape(_B * _F)
    x_emb = _build_sc_gather()(emb_lin, cat_flat).reshape(_B, _F * _D)
    w0e = W0[:, : _F * _D]
    w0c = W0[:, _F * _D:]
    w3p = jnp.pad(W3, ((0, 127), (0, 0)))
    return _mlp(x_emb, cont_data,
                w0e, w0c, b0.reshape(1, -1),
                W1, b1.reshape(1, -1),
                W2, b2.reshape(1, -1),
                w3p, b3.reshape(1, -1))


# speedup vs baseline: 2.6875x; 1.0018x over previous
import functools

import jax
import jax.numpy as jnp
from jax import lax
from jax.experimental import pallas as pl
from jax.experimental.pallas import tpu as pltpu
from jax.experimental.pallas import tpu_sc as plsc

_B = 4096
_F = 26
_V = 100000
_D = 32
_NC, _NS = 2, 16
_NW = _NC * _NS
_RPW = _B // _NW
_LPW = _RPW * _F
_EPW = _LPW * _D
_CH = 128
_SEG = 4 * _CH
_NSEG = 32


@functools.cache
def _build_sc_gather():
    mesh = plsc.VectorSubcoreMesh(core_axis_name="c", subcore_axis_name="s",
                                  num_cores=_NC, num_subcores=_NS)
    return functools.partial(
        pl.kernel,
        out_type=jax.ShapeDtypeStruct((_B * _F * _D,), jnp.float32),
        mesh=mesh,
        scratch_types=[
            pltpu.VMEM((_LPW,), jnp.int32),
            pltpu.VMEM((_NSEG * _SEG,), jnp.int32),
            pltpu.VMEM((_EPW,), jnp.float32),
            pltpu.SemaphoreType.DMA,
        ],
        compiler_params=pltpu.CompilerParams(use_tc_tiling_on_sc=False),
    )(_sc_gather_body)


def _sc_gather_body(emb_hbm, cat_hbm, out_hbm, cat_v, ring_v, val_v, sem):
    wid = lax.axis_index("s") * _NC + lax.axis_index("c")
    base = wid * _LPW
    pltpu.sync_copy(cat_hbm.at[pl.ds(base, _LPW)], cat_v)

    lane = lax.iota(jnp.int32, 16)
    laneV = lane * _V

    def chunk_copy(c, slot):
        return pltpu.make_async_copy(
            emb_hbm.at[ring_v.at[pl.ds(slot * _CH, _CH)]],
            val_v.at[pl.ds(c * _CH, _CH)],
            sem,
        )

    n_outer = _LPW // 16

    def outer(i, _):
        seg = lax.rem(i, _NSEG)

        @pl.when(i >= _NSEG)
        def _wait_prev():
            prev = i - _NSEG
            pseg = lax.rem(prev, _NSEG)
            for k in range(4):
                chunk_copy(prev * 4 + k, pseg * 4 + k).wait()

        cat16 = cat_v[pl.ds(i * 16, 16)]
        f16 = (i * 16 + lane) % _F
        base16 = f16 * (_D * _V) + cat16
        gdn = lax.GatherDimensionNumbers(
            offset_dims=(), collapsed_slice_dims=(0,), start_index_map=(0,))
        for j in range(16):
            bj = lax.gather(base16, jnp.full((16, 1), j, jnp.int32), gdn,
                            slice_sizes=(1,),
                            mode=lax.GatherScatterMode.PROMISE_IN_BOUNDS)
            for h in range(2):
                ring_v[pl.ds(seg * _SEG + j * _D + h * 16, 16)] = (
                    bj + laneV + h * 16 * _V)
        for k in range(4):
            pltpu.async_copy(
                emb_hbm.at[ring_v.at[pl.ds(seg * _SEG + k * _CH, _CH)]],
                val_v.at[pl.ds((i * 4 + k) * _CH, _CH)],
                sem,
            )
        return 0

    lax.fori_loop(0, n_outer, outer, 0)

    def drain(j, _):
        c = (n_outer - _NSEG) * 4 + j
        slot = lax.rem(c, _NSEG * 4)
        chunk_copy(c, slot).wait()
        return 0

    lax.fori_loop(0, _NSEG * 4, drain, 0)
    pltpu.sync_copy(val_v, out_hbm.at[pl.ds(wid * _EPW, _EPW)])


_BT = 1024


def _mlp_body(xe_ref, xc_ref, w0e_ref, w0c_ref, b0_ref, w1_ref, b1_ref,
              w2_ref, b2_ref, w3_ref, b3_ref, o_ref):
    dn = (((1,), (1,)), ((), ()))
    h = lax.dot_general(xe_ref[...], w0e_ref[...], dn,
                        preferred_element_type=jnp.float32)
    h += lax.dot_general(xc_ref[...], w0c_ref[...], dn,
                         preferred_element_type=jnp.float32)
    h = jnp.maximum(h + b0_ref[...], 0.0)
    h = lax.dot_general(h, w1_ref[...], dn, preferred_element_type=jnp.float32)
    h = jnp.maximum(h + b1_ref[...], 0.0)
    h = lax.dot_general(h, w2_ref[...], dn, preferred_element_type=jnp.float32)
    h = jnp.maximum(h + b2_ref[...], 0.0)
    o = lax.dot_general(h, w3_ref[...], dn, preferred_element_type=jnp.float32)
    o_ref[...] = jax.nn.sigmoid(o[:, :1] + b3_ref[0, 0])


def _mlp(x_emb, cont, w0e, w0c, b0, w1, b1, w2, b2, w3, b3):
    grid = (_B // _BT,)
    full = lambda shape: pl.BlockSpec(shape, lambda i: (0, 0))
    return pl.pallas_call(
        _mlp_body,
        grid=grid,
        in_specs=[
            pl.BlockSpec((_BT, _F * _D), lambda i: (i, 0)),
            pl.BlockSpec((_BT, 13), lambda i: (i, 0)),
            full(w0e.shape), full(w0c.shape), full(b0.shape),
            full(w1.shape), full(b1.shape),
            full(w2.shape), full(b2.shape),
            full(w3.shape), full(b3.shape),
        ],
        out_specs=pl.BlockSpec((_BT, 1), lambda i: (i, 0)),
        out_shape=jax.ShapeDtypeStruct((_B, 1), jnp.float32),
    )(x_emb, cont, w0e, w0c, b0, w1, b1, w2, b2, w3, b3)


def kernel(cont_data, cat_data, emb_tables, W0, b0, W1, b1, W2, b2, W3, b3):
    emb_lin = emb_tables.transpose(0, 2, 1).reshape(_F * _D * _V)
    cat_flat = cat_data.reshape(_B * _F)
    x_emb = _build_sc_gather()(emb_lin, cat_flat).reshape(_B, _F * _D)
    w0e = W0[:, : _F * _D]
    w0c = W0[:, _F * _D:]
    w3p = jnp.pad(W3, ((0, 127), (0, 0)))
    return _mlp(x_emb, cont_data,
                w0e, w0c, b0.reshape(1, -1),
                W1, b1.reshape(1, -1),
                W2, b2.reshape(1, -1),
                w3p, b3.reshape(1, -1))
